# Initial kernel scaffold; baseline (speedup 1.0000x reference)
import jax
import jax.numpy as jnp
from jax.experimental import pallas as pl


def kernel(pred_boxes, anchors, gt_boxes, gt_labels):
    raise NotImplementedError("write your pallas kernel here")


# speedup vs baseline: 40.3324x; 40.3324x over previous
import functools

import jax
import jax.numpy as jnp
from jax import lax
from jax.experimental import pallas as pl
from jax.experimental.pallas import tpu as pltpu
from jax.experimental.pallas import tpu_sc as plsc

_BS = 4
_NQ = 20000
_NGT = 50
_MT = 4
_L = 16
_NB = _NQ // _L
_NCOL = 13
_BIG = 2**30


def _lexmin(a, ai, b, bi):
    cond = (b < a) | ((b == a) & (bi < ai))
    return jnp.where(cond, b, a), jnp.where(cond, bi, ai)


def _matcher_body(src_hbm, gt_hbm, out_hbm, plane, gtv, outv):
    c = lax.axis_index("c")
    s = lax.axis_index("s")
    wid = c * 16 + s
    combo = wid // 4
    k = wid % 4
    src = combo // 4
    i = combo % 4
    r0 = jnp.where(k < 3, k * _NCOL, _NGT - _NCOL)

    pltpu.sync_copy(src_hbm.at[src, i], plane)
    pltpu.sync_copy(gt_hbm, gtv)

    def conv(b, carry):
        ds = pl.ds(b * _L, _L)
        x0 = plane[0, ds]
        y0 = plane[1, ds]
        x1 = plane[2, ds]
        y1 = plane[3, ds]
        plane[0, ds] = (x0 + x1) * 0.5
        plane[1, ds] = (y0 + y1) * 0.5
        plane[2, ds] = x1 - x0
        plane[3, ds] = y1 - y0
        return carry

    lax.fori_loop(0, _NB, conv, 0)

    inf = jnp.float32(jnp.inf)
    lane = lax.iota(jnp.int32, _L)

    def col_body(j, carry):
        r = r0 + j
        grow = gtv[pl.ds((i * _NGT + r) * 4, _L)]
        gx0 = grow[0]
        gy0 = grow[1]
        gx1 = grow[2]
        gy1 = grow[3]
        gcx = (gx0 + gx1) * 0.5
        gcy = (gy0 + gy1) * 0.5
        gw = gx1 - gx0
        gh = gy1 - gy0

        m0 = jnp.full((_L,), inf, jnp.float32)
        z = jnp.zeros((_L,), jnp.int32)
        ci0 = lax.iota(jnp.int32, _L)

        def blk(b, st):
            m1, m2, m3, m4, i1, i2, i3, i4, ci = st
            ds = pl.ds(b * _L, _L)
            cost = (jnp.abs(plane[0, ds] - gcx)
                    + jnp.abs(plane[1, ds] - gcy)
                    + jnp.abs(plane[2, ds] - gw)
                    + jnp.abs(plane[3, ds] - gh))
            cv, cvi = cost, ci
            cnd = cv < m1
            m1n = jnp.where(cnd, cv, m1)
            i1n = jnp.where(cnd, cvi, i1)
            cv, cvi = jnp.where(cnd, m1, cv), jnp.where(cnd, i1, cvi)
            cnd = cv < m2
            m2n = jnp.where(cnd, cv, m2)
            i2n = jnp.where(cnd, cvi, i2)
            cv, cvi = jnp.where(cnd, m2, cv), jnp.where(cnd, i2, cvi)
            cnd = cv < m3
            m3n = jnp.where(cnd, cv, m3)
            i3n = jnp.where(cnd, cvi, i3)
            cv, cvi = jnp.where(cnd, m3, cv), jnp.where(cnd, i3, cvi)
            cnd = cv < m4
            m4n = jnp.where(cnd, cv, m4)
            i4n = jnp.where(cnd, cvi, i4)
            return (m1n, m2n, m3n, m4n, i1n, i2n, i3n, i4n, ci + _L)

        st = lax.fori_loop(0, _NB, blk,
                           (m0, m0, m0, m0, z, z, z, z, ci0))
        m = [st[0], st[1], st[2], st[3]]
        mi = [st[4], st[5], st[6], st[7]]

        os = list(carry)
        for t in range(_MT):
            v, vi = _lexmin(m[0], mi[0], m[1], mi[1])
            w, wi = _lexmin(m[2], mi[2], m[3], mi[3])
            v, vi = _lexmin(v, vi, w, wi)
            sv = jnp.min(v)
            im = jnp.where(v == sv, vi, jnp.int32(_BIG))
            si = jnp.min(im)
            os[t] = jnp.where(lane == j, si, os[t])
            for lvl in range(_MT):
                hit = (m[lvl] == sv) & (mi[lvl] == si)
                m[lvl] = jnp.where(hit, inf, m[lvl])
        return tuple(os)

    z16 = jnp.zeros((_L,), jnp.int32)
    orows = lax.fori_loop(0, _NCOL, col_body, (z16, z16, z16, z16))
    for t in range(_MT):
        outv[t, :] = orows[t]

    pltpu.sync_copy(outv, out_hbm.at[wid])


@functools.partial(
    pl.kernel,
    out_type=jax.ShapeDtypeStruct((32, _MT, _L), jnp.int32),
    mesh=plsc.VectorSubcoreMesh(core_axis_name="c", subcore_axis_name="s"),
    compiler_params=pltpu.CompilerParams(needs_layout_passes=False),
    scratch_types=[
        pltpu.VMEM((4, _NQ), jnp.float32),
        pltpu.VMEM((_BS * _NGT * 4 + 2 * _L,), jnp.float32),
        pltpu.VMEM((_MT, _L), jnp.int32),
    ],
)
def _matcher(src_hbm, gt_hbm, out_hbm, plane, gtv, outv):
    _matcher_body(src_hbm, gt_hbm, out_hbm, plane, gtv, outv)


def kernel(pred_boxes, anchors, gt_boxes, gt_labels):
    bs, nq = pred_boxes.shape[:2]
    ngt = gt_boxes.shape[1]

    pp = pred_boxes.reshape(nq, bs, 4).transpose(1, 2, 0)
    ap = anchors.reshape(nq, bs, 4).transpose(1, 2, 0)
    src_t = jnp.stack([pp, ap])
    gt_flat = jnp.concatenate(
        [gt_boxes.reshape(-1), jnp.zeros((2 * _L,), jnp.float32)])

    out = _matcher(src_t, gt_flat)

    o = out.reshape(2, bs, 4, _MT, _L)
    full = jnp.concatenate(
        [o[:, :, 0, :, :_NCOL],
         o[:, :, 1, :, :_NCOL],
         o[:, :, 2, :, :_NCOL],
         o[:, :, 3, :, 3 * _NCOL - (_NGT - _NCOL):_NCOL]],
        axis=-1)
    idx_i = full.transpose(1, 2, 0, 3).reshape(bs, _MT * 2 * ngt)

    base_j = jnp.tile(
        jnp.concatenate([jnp.arange(ngt, dtype=jnp.int32)] * 2), _MT)
    idx_j = jnp.broadcast_to(base_j, (bs, base_j.shape[0]))
    return idx_i, idx_j
